# Initial kernel scaffold; baseline (speedup 1.0000x reference)
import jax
import jax.numpy as jnp
from jax.experimental import pallas as pl


def kernel(directions, perturbations, W_in0, W_in1, W_hid0_log, w_out_log, length_scale):
    raise NotImplementedError("write your pallas kernel here")


# speedup vs baseline: 2.4915x; 2.4915x over previous
import functools

import jax
import jax.numpy as jnp
from jax.experimental import pallas as pl

_C = 5
_K = 4
_W = 256
_BB = 2048
_DEF = jax.lax.Precision.DEFAULT


def _fused_kernel(d_ref, pert_ref, w0_ref, w1_ref, h_ref, wl_ref, out_ref):
    f32 = jnp.float32
    H = h_ref[...]
    wL = wl_ref[...]
    W0 = w0_ref[...]
    W1 = w1_ref[...]

    d0 = d_ref[:, 0:1]
    d1 = d_ref[:, 1:2]
    d2 = d_ref[:, 2:3]

    dots = []
    verts = []
    for c in range(_C):
        ux = d0 + pert_ref[c:c + 1, 0:1]
        uy = d1 + pert_ref[c:c + 1, 1:2]
        uz = d2 + pert_ref[c:c + 1, 2:3]
        nrm = jnp.sqrt(ux * ux + uy * uy + uz * uz)
        X = jnp.concatenate([ux / nrm, uy / nrm, uz / nrm], axis=1)

        z1 = jnp.dot(X, W0, preferred_element_type=f32, precision=_DEF)
        h1 = jnp.maximum(z1, 0.0)
        z2 = (jnp.dot(h1, H, preferred_element_type=f32, precision=_DEF)
              + jnp.dot(X, W1, preferred_element_type=f32, precision=_DEF))
        g2 = jnp.where(z2 > 0.0, wL, 0.0)
        t = jax.lax.dot_general(g2, H, (((1,), (1,)), ((), ())),
                                preferred_element_type=f32,
                                precision=_DEF)
        g1 = jnp.where(z1 > 0.0, t, 0.0)
        v = (jax.lax.dot_general(g2, W1, (((1,), (1,)), ((), ())),
                                 preferred_element_type=f32, precision=_DEF)
             + jax.lax.dot_general(g1, W0, (((1,), (1,)), ((), ())),
                                   preferred_element_type=f32,
                                   precision=_DEF))
        vx = v[:, 0:1]
        vy = v[:, 1:2]
        vz = v[:, 2:3]
        verts.append((vx, vy, vz))
        dots.append(d0 * vx + d1 * vy + d2 * vz)

    ranks = []
    for c in range(_C):
        r = jnp.zeros_like(dots[c], dtype=jnp.int32)
        for j in range(_C):
            if j == c:
                continue
            beats = dots[j] > dots[c]
            if j < c:
                beats = beats | (dots[j] == dots[c])
            r = r + beats.astype(jnp.int32)
        ranks.append(r)

    cols = []
    for r in range(_K):
        ox = jnp.zeros_like(d0)
        oy = jnp.zeros_like(d0)
        oz = jnp.zeros_like(d0)
        for c in range(_C):
            sel = (ranks[c] == r).astype(jnp.float32)
            vx, vy, vz = verts[c]
            ox = ox + sel * vx
            oy = oy + sel * vy
            oz = oz + sel * vz
        cols.extend([ox, oy, oz])
    out_ref[...] = jnp.concatenate(cols, axis=1)


@jax.jit
def _run(directions, pert_full, W_in0, W_in1, H, wL):
    B = directions.shape[0]
    grid = (B // _BB,)
    out = pl.pallas_call(
        _fused_kernel,
        grid=grid,
        in_specs=[
            pl.BlockSpec((_BB, 3), lambda i: (i, 0)),
            pl.BlockSpec((_C, 3), lambda i: (0, 0)),
            pl.BlockSpec((3, _W), lambda i: (0, 0)),
            pl.BlockSpec((3, _W), lambda i: (0, 0)),
            pl.BlockSpec((_W, _W), lambda i: (0, 0)),
            pl.BlockSpec((1, _W), lambda i: (0, 0)),
        ],
        out_specs=pl.BlockSpec((_BB, 3 * _K), lambda i: (i, 0)),
        out_shape=jax.ShapeDtypeStruct((B, 3 * _K), jnp.float32),
    )(directions, pert_full, W_in0, W_in1, H, wL)
    return out.reshape(B, _K, 3)


def kernel(directions, perturbations, W_in0, W_in1, W_hid0_log, w_out_log,
           length_scale):
    pert_full = jnp.concatenate(
        [jnp.zeros((1, 3), directions.dtype), perturbations], axis=0)
    H = jnp.exp(W_hid0_log)
    wL = (jnp.exp(w_out_log) * length_scale).reshape(1, _W)
    return _run(directions, pert_full, W_in0, W_in1, H, wL)


# speedup vs baseline: 3.7487x; 1.5046x over previous
import functools

import jax
import jax.numpy as jnp
from jax.experimental import pallas as pl

_C = 5
_K = 4
_W = 256
_BB = 2048
_DEF = jax.lax.Precision.DEFAULT


def _fused_kernel(d_ref, pert_ref, w0_ref, w1_ref, h_ref, wl_ref, out_ref):
    f32 = jnp.float32
    H = h_ref[...]
    wL = wl_ref[...]
    W0 = w0_ref[...]
    W1 = w1_ref[...]

    D = d_ref[...]

    dots = []
    verts = []
    for c in range(_C):
        U = D + pert_ref[c:c + 1, :]
        nrm = jnp.sqrt(jnp.sum(U * U, axis=1, keepdims=True))
        X = U / nrm

        z1 = jnp.dot(X, W0, preferred_element_type=f32, precision=_DEF)
        h1 = jnp.maximum(z1, 0.0)
        z2 = (jnp.dot(h1, H, preferred_element_type=f32, precision=_DEF)
              + jnp.dot(X, W1, preferred_element_type=f32, precision=_DEF))
        g2 = jnp.where(z2 > 0.0, wL, 0.0)
        t = jax.lax.dot_general(g2, H, (((1,), (1,)), ((), ())),
                                preferred_element_type=f32,
                                precision=_DEF)
        g1 = jnp.where(z1 > 0.0, t, 0.0)
        v = (jax.lax.dot_general(g2, W1, (((1,), (1,)), ((), ())),
                                 preferred_element_type=f32, precision=_DEF)
             + jax.lax.dot_general(g1, W0, (((1,), (1,)), ((), ())),
                                   preferred_element_type=f32,
                                   precision=_DEF))
        verts.append(v)
        dots.append(jnp.sum(D * v, axis=1, keepdims=True))

    ranks = []
    for c in range(_C):
        r = jnp.zeros_like(dots[c], dtype=jnp.int32)
        for j in range(_C):
            if j == c:
                continue
            beats = dots[j] > dots[c]
            if j < c:
                beats = beats | (dots[j] == dots[c])
            r = r + beats.astype(jnp.int32)
        ranks.append(r)

    for r in range(_K):
        o = jnp.zeros_like(verts[0])
        for c in range(_C):
            sel = (ranks[c] == r).astype(jnp.float32)
            o = o + sel * verts[c]
        out_ref[:, 3 * r:3 * r + 3] = o


@jax.jit
def _run(directions, pert_full, W_in0, W_in1, H, wL):
    B = directions.shape[0]
    grid = (B // _BB,)
    out = pl.pallas_call(
        _fused_kernel,
        grid=grid,
        in_specs=[
            pl.BlockSpec((_BB, 3), lambda i: (i, 0)),
            pl.BlockSpec((_C, 3), lambda i: (0, 0)),
            pl.BlockSpec((3, _W), lambda i: (0, 0)),
            pl.BlockSpec((3, _W), lambda i: (0, 0)),
            pl.BlockSpec((_W, _W), lambda i: (0, 0)),
            pl.BlockSpec((1, _W), lambda i: (0, 0)),
        ],
        out_specs=pl.BlockSpec((_BB, 3 * _K), lambda i: (i, 0)),
        out_shape=jax.ShapeDtypeStruct((B, 3 * _K), jnp.float32),
    )(directions, pert_full, W_in0, W_in1, H, wL)
    return out.reshape(B, _K, 3)


def kernel(directions, perturbations, W_in0, W_in1, W_hid0_log, w_out_log,
           length_scale):
    pert_full = jnp.concatenate(
        [jnp.zeros((1, 3), directions.dtype), perturbations], axis=0)
    H = jnp.exp(W_hid0_log)
    wL = (jnp.exp(w_out_log) * length_scale).reshape(1, _W)
    return _run(directions, pert_full, W_in0, W_in1, H, wL)


# speedup vs baseline: 6.8570x; 1.8292x over previous
import jax
import jax.numpy as jnp
from jax.experimental import pallas as pl

_C = 5
_K = 4
_W = 256
_BB = 1024
_DEF = jax.lax.Precision.DEFAULT


def _fused_kernel(dt_ref, pert_ref, w01_ref, w0_ref, w1_ref, h_ref, wl_ref,
                  out_ref):
    f32 = jnp.float32
    H = h_ref[...]
    wL = wl_ref[...]
    DT = dt_ref[...]

    UT = jnp.concatenate(
        [DT + pert_ref[:, c:c + 1] for c in range(_C)], axis=1)
    nrm = jnp.sqrt(jnp.sum(UT * UT, axis=0, keepdims=True))
    XT = UT / nrm

    zz = jax.lax.dot_general(w01_ref[...], XT, (((0,), (0,)), ((), ())),
                             preferred_element_type=f32, precision=_DEF)
    z1 = zz[0:_W, :]
    h1 = jnp.maximum(z1, 0.0)
    z2 = jax.lax.dot_general(H, h1, (((0,), (0,)), ((), ())),
                             preferred_element_type=f32,
                             precision=_DEF) + zz[_W:2 * _W, :]
    g2 = jnp.where(z2 > 0.0, wL, 0.0)
    t = jax.lax.dot_general(H, g2, (((1,), (0,)), ((), ())),
                            preferred_element_type=f32, precision=_DEF)
    g1 = jnp.where(z1 > 0.0, t, 0.0)
    v = (jax.lax.dot_general(w1_ref[...], g2, (((1,), (0,)), ((), ())),
                             preferred_element_type=f32, precision=_DEF)
         + jax.lax.dot_general(w0_ref[...], g1, (((1,), (0,)), ((), ())),
                               preferred_element_type=f32,
                               precision=_DEF))

    d0 = DT[0:1, :]
    d1 = DT[1:2, :]
    d2 = DT[2:3, :]
    verts = []
    dots = []
    for c in range(_C):
        vc = v[:, c * _BB:(c + 1) * _BB]
        verts.append(vc)
        dots.append(d0 * vc[0:1, :] + d1 * vc[1:2, :]
                    + d2 * vc[2:3, :])

    ranks = []
    for c in range(_C):
        r = jnp.zeros_like(dots[c], dtype=jnp.int32)
        for j in range(_C):
            if j == c:
                continue
            beats = dots[j] > dots[c]
            if j < c:
                beats = beats | (dots[j] == dots[c])
            r = r + beats.astype(jnp.int32)
        ranks.append(r)

    for r in range(_K):
        o = jnp.zeros_like(verts[0])
        for c in range(_C):
            sel = (ranks[c] == r).astype(jnp.float32)
            o = o + sel * verts[c]
        out_ref[3 * r:3 * r + 3, :] = o


@jax.jit
def _run(directions_t, pert_t, W01, W_in0, W_in1, H, wL):
    B = directions_t.shape[1]
    grid = (B // _BB,)
    out = pl.pallas_call(
        _fused_kernel,
        grid=grid,
        in_specs=[
            pl.BlockSpec((3, _BB), lambda i: (0, i)),
            pl.BlockSpec((3, _C), lambda i: (0, 0)),
            pl.BlockSpec((3, 2 * _W), lambda i: (0, 0)),
            pl.BlockSpec((3, _W), lambda i: (0, 0)),
            pl.BlockSpec((3, _W), lambda i: (0, 0)),
            pl.BlockSpec((_W, _W), lambda i: (0, 0)),
            pl.BlockSpec((_W, 1), lambda i: (0, 0)),
        ],
        out_specs=pl.BlockSpec((3 * _K, _BB), lambda i: (0, i)),
        out_shape=jax.ShapeDtypeStruct((3 * _K, B), jnp.float32),
    )(directions_t, pert_t, W01, W_in0, W_in1, H, wL)
    return out.T.reshape(B, _K, 3)


def kernel(directions, perturbations, W_in0, W_in1, W_hid0_log, w_out_log,
           length_scale):
    pert_full = jnp.concatenate(
        [jnp.zeros((1, 3), directions.dtype), perturbations], axis=0)
    H = jnp.exp(W_hid0_log)
    wL = (jnp.exp(w_out_log) * length_scale).reshape(_W, 1)
    W01 = jnp.concatenate([W_in0, W_in1], axis=1)
    return _run(directions.T, pert_full.T, W01, W_in0, W_in1, H, wL)


# speedup vs baseline: 7.0899x; 1.0340x over previous
import jax
import jax.numpy as jnp
from jax.experimental import pallas as pl

_C = 5
_K = 4
_W = 256
_BB = 2048
_DEF = jax.lax.Precision.DEFAULT


def _fused_kernel(dt_ref, pert_ref, w01_ref, w0_ref, w1_ref, h_ref, wl_ref,
                  out_ref):
    f32 = jnp.float32
    H = h_ref[...]
    wL = wl_ref[...]
    DT = dt_ref[...]

    UT = jnp.concatenate(
        [DT + pert_ref[:, c:c + 1] for c in range(_C)], axis=1)
    nrm = jnp.sqrt(jnp.sum(UT * UT, axis=0, keepdims=True))
    XT = UT / nrm

    zz = jax.lax.dot_general(w01_ref[...], XT, (((0,), (0,)), ((), ())),
                             preferred_element_type=f32, precision=_DEF)
    z1 = zz[0:_W, :]
    h1 = jnp.maximum(z1, 0.0)
    z2 = jax.lax.dot_general(H, h1, (((0,), (0,)), ((), ())),
                             preferred_element_type=f32,
                             precision=_DEF) + zz[_W:2 * _W, :]
    g2 = jnp.where(z2 > 0.0, wL, 0.0)
    t = jax.lax.dot_general(H, g2, (((1,), (0,)), ((), ())),
                            preferred_element_type=f32, precision=_DEF)
    g1 = jnp.where(z1 > 0.0, t, 0.0)
    v = (jax.lax.dot_general(w1_ref[...], g2, (((1,), (0,)), ((), ())),
                             preferred_element_type=f32, precision=_DEF)
         + jax.lax.dot_general(w0_ref[...], g1, (((1,), (0,)), ((), ())),
                               preferred_element_type=f32,
                               precision=_DEF))

    d0 = DT[0:1, :]
    d1 = DT[1:2, :]
    d2 = DT[2:3, :]
    verts = []
    dots = []
    for c in range(_C):
        vc = v[:, c * _BB:(c + 1) * _BB]
        verts.append(vc)
        dots.append(d0 * vc[0:1, :] + d1 * vc[1:2, :]
                    + d2 * vc[2:3, :])

    ranks = []
    for c in range(_C):
        r = jnp.zeros_like(dots[c], dtype=jnp.int32)
        for j in range(_C):
            if j == c:
                continue
            beats = dots[j] > dots[c]
            if j < c:
                beats = beats | (dots[j] == dots[c])
            r = r + beats.astype(jnp.int32)
        ranks.append(r)

    for r in range(_K):
        o = jnp.zeros_like(verts[0])
        for c in range(_C):
            sel = (ranks[c] == r).astype(jnp.float32)
            o = o + sel * verts[c]
        out_ref[3 * r:3 * r + 3, :] = o


@jax.jit
def _run(directions_t, pert_t, W01, W_in0, W_in1, H, wL):
    B = directions_t.shape[1]
    grid = (B // _BB,)
    out = pl.pallas_call(
        _fused_kernel,
        grid=grid,
        in_specs=[
            pl.BlockSpec((3, _BB), lambda i: (0, i)),
            pl.BlockSpec((3, _C), lambda i: (0, 0)),
            pl.BlockSpec((3, 2 * _W), lambda i: (0, 0)),
            pl.BlockSpec((3, _W), lambda i: (0, 0)),
            pl.BlockSpec((3, _W), lambda i: (0, 0)),
            pl.BlockSpec((_W, _W), lambda i: (0, 0)),
            pl.BlockSpec((_W, 1), lambda i: (0, 0)),
        ],
        out_specs=pl.BlockSpec((3 * _K, _BB), lambda i: (0, i)),
        out_shape=jax.ShapeDtypeStruct((3 * _K, B), jnp.float32),
    )(directions_t, pert_t, W01, W_in0, W_in1, H, wL)
    return out.T.reshape(B, _K, 3)


def kernel(directions, perturbations, W_in0, W_in1, W_hid0_log, w_out_log,
           length_scale):
    pert_full = jnp.concatenate(
        [jnp.zeros((1, 3), directions.dtype), perturbations], axis=0)
    H = jnp.exp(W_hid0_log)
    wL = (jnp.exp(w_out_log) * length_scale).reshape(_W, 1)
    W01 = jnp.concatenate([W_in0, W_in1], axis=1)
    return _run(directions.T, pert_full.T, W01, W_in0, W_in1, H, wL)


# speedup vs baseline: 7.1087x; 1.0026x over previous
import jax
import jax.numpy as jnp
from jax.experimental import pallas as pl
from jax.experimental.pallas import tpu as pltpu

_C = 5
_K = 4
_W = 256
_BB = 2048
_DEF = jax.lax.Precision.DEFAULT


def _fused_kernel(dt_ref, pert_ref, w01_ref, w0_ref, w1_ref, h_ref, wl_ref,
                  out_ref):
    f32 = jnp.float32
    H = h_ref[...]
    wL = wl_ref[...]
    DT = dt_ref[...]

    UT = jnp.concatenate(
        [DT + pert_ref[:, c:c + 1] for c in range(_C)], axis=1)
    nrm = jnp.sqrt(jnp.sum(UT * UT, axis=0, keepdims=True))
    XT = UT / nrm

    zz = jax.lax.dot_general(w01_ref[...], XT, (((0,), (0,)), ((), ())),
                             preferred_element_type=f32, precision=_DEF)
    z1 = zz[0:_W, :]
    h1 = jnp.maximum(z1, 0.0)
    z2 = jax.lax.dot_general(H, h1, (((0,), (0,)), ((), ())),
                             preferred_element_type=f32,
                             precision=_DEF) + zz[_W:2 * _W, :]
    g2 = jnp.where(z2 > 0.0, wL, 0.0)
    t = jax.lax.dot_general(H, g2, (((1,), (0,)), ((), ())),
                            preferred_element_type=f32, precision=_DEF)
    g1 = jnp.where(z1 > 0.0, t, 0.0)
    v = (jax.lax.dot_general(w1_ref[...], g2, (((1,), (0,)), ((), ())),
                             preferred_element_type=f32, precision=_DEF)
         + jax.lax.dot_general(w0_ref[...], g1, (((1,), (0,)), ((), ())),
                               preferred_element_type=f32,
                               precision=_DEF))

    d0 = DT[0:1, :]
    d1 = DT[1:2, :]
    d2 = DT[2:3, :]
    verts = []
    dots = []
    for c in range(_C):
        vc = v[:, c * _BB:(c + 1) * _BB]
        verts.append(vc)
        dots.append(d0 * vc[0:1, :] + d1 * vc[1:2, :]
                    + d2 * vc[2:3, :])

    ranks = []
    for c in range(_C):
        r = jnp.zeros_like(dots[c], dtype=jnp.int32)
        for j in range(_C):
            if j == c:
                continue
            beats = dots[j] > dots[c]
            if j < c:
                beats = beats | (dots[j] == dots[c])
            r = r + beats.astype(jnp.int32)
        ranks.append(r)

    for r in range(_K):
        o = jnp.zeros_like(verts[0])
        for c in range(_C):
            sel = (ranks[c] == r).astype(jnp.float32)
            o = o + sel * verts[c]
        out_ref[3 * r:3 * r + 3, :] = o


@jax.jit
def _run(directions_t, pert_t, W01, W_in0, W_in1, H, wL):
    B = directions_t.shape[1]
    grid = (B // _BB,)
    out = pl.pallas_call(
        _fused_kernel,
        grid=grid,
        in_specs=[
            pl.BlockSpec((3, _BB), lambda i: (0, i)),
            pl.BlockSpec((3, _C), lambda i: (0, 0)),
            pl.BlockSpec((3, 2 * _W), lambda i: (0, 0)),
            pl.BlockSpec((3, _W), lambda i: (0, 0)),
            pl.BlockSpec((3, _W), lambda i: (0, 0)),
            pl.BlockSpec((_W, _W), lambda i: (0, 0)),
            pl.BlockSpec((_W, 1), lambda i: (0, 0)),
        ],
        out_specs=pl.BlockSpec((3 * _K, _BB), lambda i: (0, i)),
        out_shape=jax.ShapeDtypeStruct((3 * _K, B), jnp.float32),
        compiler_params=pltpu.CompilerParams(
            dimension_semantics=("parallel",)),
    )(directions_t, pert_t, W01, W_in0, W_in1, H, wL)
    return out.T.reshape(B, _K, 3)


def kernel(directions, perturbations, W_in0, W_in1, W_hid0_log, w_out_log,
           length_scale):
    pert_full = jnp.concatenate(
        [jnp.zeros((1, 3), directions.dtype), perturbations], axis=0)
    H = jnp.exp(W_hid0_log)
    wL = (jnp.exp(w_out_log) * length_scale).reshape(_W, 1)
    W01 = jnp.concatenate([W_in0, W_in1], axis=1)
    return _run(directions.T, pert_full.T, W01, W_in0, W_in1, H, wL)
